# baseline (speedup 1.0000x reference)
import functools

import jax
import numpy as np
import jax.numpy as jnp
from jax import lax
from jax.experimental import pallas as pl
from jax.experimental.pallas import tpu as pltpu
from jax.experimental.pallas import tpu_sc as plsc

N = 10000
E = 320000
D = 128
K = 2

NC = 2
NS = 16
NW = NC * NS
EW = E // NW
C = 128
NCHUNK = 80
EWP = NCHUNK * C
EP = NW * EWP
NP = 10240
RPW = NP // NS
DEGW = 16

_mesh = plsc.VectorSubcoreMesh(core_axis_name="c", subcore_axis_name="s")


@functools.partial(
    pl.kernel,
    out_type=jax.ShapeDtypeStruct((NW, NP), jnp.float32),
    mesh=_mesh,
    compiler_params=pltpu.CompilerParams(needs_layout_passes=False),
    scratch_types=[
        pltpu.VMEM((NCHUNK, C), jnp.int32),
        pltpu.VMEM((NP,), jnp.float32),
    ],
)
def _sc_deg(dst_hbm, zerosflat_hbm, out_hbm, dst_v, deg_v):
    c = lax.axis_index("c")
    s = lax.axis_index("s")
    wid = c * NS + s
    pltpu.sync_copy(dst_hbm.at[wid], dst_v)
    pltpu.sync_copy(zerosflat_hbm, deg_v)
    ones16 = jnp.ones((16,), jnp.float32)

    def body(i, _):
        for t in range(C // 16):
            ids = dst_v[i, pl.ds(t * 16, 16)]
            plsc.addupdate_scatter(deg_v, [ids], ones16)
        return 0

    lax.fori_loop(0, NCHUNK, body, 0)
    pltpu.sync_copy(deg_v, out_hbm.at[wid])


IB = 4
NBLK = NCHUNK // IB


@functools.partial(
    pl.kernel,
    out_type=jax.ShapeDtypeStruct((NC, NP, D), jnp.float32),
    mesh=_mesh,
    scratch_types=[
        pltpu.VMEM((2, IB, C), jnp.int32),
        pltpu.VMEM((2, IB, C), jnp.int32),
        pltpu.VMEM((2, C, D), jnp.float32),
        pltpu.SemaphoreType.DMA((2,)),
        pltpu.SemaphoreType.DMA((2,)),
        pltpu.VMEM_SHARED((NP, D), jnp.float32),
    ],
)
def _sc_hop(rst_hbm, src_hbm, dst_hbm, zeros_hbm, out_hbm,
            sidx_v, didx_v, rows_v, isem, gsem, agg_sp):
    c = lax.axis_index("c")
    s = lax.axis_index("s")
    wid = c * NS + s
    pltpu.sync_copy(zeros_hbm, agg_sp.at[pl.ds(s * RPW, RPW)])
    pltpu.sync_copy(src_hbm.at[wid, 0], sidx_v.at[0])
    pltpu.sync_copy(dst_hbm.at[wid, 0], didx_v.at[0])
    pltpu.async_copy(src_hbm.at[wid, 1], sidx_v.at[1], isem.at[1])
    pltpu.async_copy(dst_hbm.at[wid, 1], didx_v.at[1], isem.at[1])
    plsc.subcore_barrier()

    pltpu.async_copy(rst_hbm.at[sidx_v.at[0, 0]], rows_v.at[0], gsem.at[0])
    pltpu.async_copy(rst_hbm.at[sidx_v.at[0, 1]], rows_v.at[1], gsem.at[1])

    def body(i, _):
        b = i // IB
        r = lax.rem(i, IB)
        jb = lax.rem(b, 2)
        j = lax.rem(i, 2)
        pltpu.make_async_copy(rst_hbm.at[sidx_v.at[jb, r]], rows_v.at[j],
                              gsem.at[j]).wait()
        pltpu.sync_copy(rows_v.at[j], agg_sp.at[didx_v.at[jb, r]],
                        add=True)

        @pl.when(jnp.logical_and(r == IB - 2, b + 1 < NBLK))
        def _():
            nb = lax.rem(b + 1, 2)
            pltpu.make_async_copy(src_hbm.at[wid, b + 1], sidx_v.at[nb],
                                  isem.at[nb]).wait()
            pltpu.make_async_copy(dst_hbm.at[wid, b + 1], didx_v.at[nb],
                                  isem.at[nb]).wait()

        @pl.when(i + 2 < NCHUNK)
        def _():
            i2 = i + 2
            pltpu.async_copy(rst_hbm.at[sidx_v.at[lax.rem(i2 // IB, 2),
                                                  lax.rem(i2, IB)]],
                             rows_v.at[j], gsem.at[j])

        @pl.when(jnp.logical_and(r == IB - 1, b + 2 < NBLK))
        def _():
            pltpu.async_copy(src_hbm.at[wid, b + 2], sidx_v.at[jb],
                             isem.at[jb])
            pltpu.async_copy(dst_hbm.at[wid, b + 2], didx_v.at[jb],
                             isem.at[jb])

        return 0

    lax.fori_loop(0, NCHUNK, body, 0)
    plsc.subcore_barrier()
    pltpu.sync_copy(agg_sp.at[pl.ds(s * RPW, RPW)],
                    out_hbm.at[c].at[pl.ds(s * RPW, RPW)])


DR = NP // D


def _degsum_body(dp_ref, out_ref):
    out_ref[...] = jnp.sum(dp_ref[...], axis=0)


def _tc_degsum(dp):
    return pl.pallas_call(
        _degsum_body,
        out_shape=jax.ShapeDtypeStruct((DR, D), jnp.float32),
    )(dp.reshape(NW, DR, D))


_R = 1000


def _prep_body(deg_ref, feat_ref, norm_ref, rst_ref):
    nrm = lax.rsqrt(jnp.maximum(deg_ref[...], 1.0))
    norm_ref[...] = nrm
    rst_ref[...] = feat_ref[...] * nrm


def _tc_prep(deg, feat):
    return pl.pallas_call(
        _prep_body,
        grid=(N // _R,),
        in_specs=[
            pl.BlockSpec((_R, 1), lambda i: (i, 0)),
            pl.BlockSpec((_R, D), lambda i: (i, 0)),
        ],
        out_specs=[
            pl.BlockSpec((_R, 1), lambda i: (i, 0)),
            pl.BlockSpec((_R, D), lambda i: (i, 0)),
        ],
        out_shape=[
            jax.ShapeDtypeStruct((N, 1), jnp.float32),
            jax.ShapeDtypeStruct((N, D), jnp.float32),
        ],
    )(deg, feat)


def _mid_body(p0_ref, p1_ref, norm_ref, h_ref, rst_ref):
    nrm = norm_ref[...]
    h = (p0_ref[0] + p1_ref[0]) * nrm
    h_ref[...] = h
    rst_ref[...] = h * nrm


def _tc_mid(p, norm):
    return pl.pallas_call(
        _mid_body,
        grid=(N // _R,),
        in_specs=[
            pl.BlockSpec((1, _R, D), lambda i: (0, i, 0)),
            pl.BlockSpec((1, _R, D), lambda i: (1, i, 0)),
            pl.BlockSpec((_R, 1), lambda i: (i, 0)),
        ],
        out_specs=[
            pl.BlockSpec((_R, D), lambda i: (i, 0)),
            pl.BlockSpec((_R, D), lambda i: (i, 0)),
        ],
        out_shape=[
            jax.ShapeDtypeStruct((N, D), jnp.float32),
            jax.ShapeDtypeStruct((N, D), jnp.float32),
        ],
    )(p, p, norm)


def _final_body(p0_ref, p1_ref, norm_ref, feat_ref, h1_ref,
                w0_ref, w1_ref, w2_ref, b_ref, out_ref):
    h2 = (p0_ref[0] + p1_ref[0]) * norm_ref[...]
    acc = jnp.dot(feat_ref[...], w0_ref[...],
                  preferred_element_type=jnp.float32)
    acc += jnp.dot(h1_ref[...], w1_ref[...],
                   preferred_element_type=jnp.float32)
    acc += jnp.dot(h2, w2_ref[...], preferred_element_type=jnp.float32)
    out_ref[...] = acc + b_ref[...]


def _tc_final(p, norm, feat, h1, w0, w1, w2, b2d):
    OUT = w0.shape[1]
    return pl.pallas_call(
        _final_body,
        grid=(N // _R,),
        in_specs=[
            pl.BlockSpec((1, _R, D), lambda i: (0, i, 0)),
            pl.BlockSpec((1, _R, D), lambda i: (1, i, 0)),
            pl.BlockSpec((_R, 1), lambda i: (i, 0)),
            pl.BlockSpec((_R, D), lambda i: (i, 0)),
            pl.BlockSpec((_R, D), lambda i: (i, 0)),
            pl.BlockSpec((D, OUT), lambda i: (0, 0)),
            pl.BlockSpec((D, OUT), lambda i: (0, 0)),
            pl.BlockSpec((D, OUT), lambda i: (0, 0)),
            pl.BlockSpec((1, OUT), lambda i: (0, 0)),
        ],
        out_specs=pl.BlockSpec((_R, OUT), lambda i: (i, 0)),
        out_shape=jax.ShapeDtypeStruct((N, OUT), jnp.float32),
    )(p, p, norm, feat, h1, w0, w1, w2, b2d)


def kernel(feat, edge_index, W, b):
    npad = EP - E
    pad_src = jnp.asarray(np.arange(npad, dtype=np.int32) % N)
    pad_dst = jnp.asarray(N + np.arange(npad, dtype=np.int32) % (NP - N))
    src = jnp.concatenate([edge_index[0], pad_src]).reshape(NW, NBLK, IB, C)
    dst = jnp.concatenate([edge_index[1], pad_dst]).reshape(NW, NBLK, IB, C)
    zerosflat = jnp.zeros((NP,), jnp.float32)
    zeros = jnp.zeros((RPW, D), jnp.float32)

    dp = _sc_deg(dst.reshape(NW, NCHUNK, C), zerosflat)
    deg = _tc_degsum(dp).reshape(NP, 1)[:N]
    norm, rst = _tc_prep(deg, feat)

    p = _sc_hop(rst, src, dst, zeros)
    h1, rst1 = _tc_mid(p, norm)

    p2 = _sc_hop(rst1, src, dst, zeros)

    w0, w1, w2 = W[:D], W[D:2 * D], W[2 * D:]
    return _tc_final(p2, norm, feat, h1, w0, w1, w2,
                     b.reshape(1, -1))


# speedup vs baseline: 1.0488x; 1.0488x over previous
import functools

import jax
import numpy as np
import jax.numpy as jnp
from jax import lax
from jax.experimental import pallas as pl
from jax.experimental.pallas import tpu as pltpu
from jax.experimental.pallas import tpu_sc as plsc

N = 10000
E = 320000
D = 128
K = 2

NC = 2
NS = 16
NW = NC * NS
EW = E // NW
C = 112
NCHUNK = 90
EWP = NCHUNK * C
EP = NW * EWP
NP = 10240
RPW = NP // NS
DEGW = 16

_mesh = plsc.VectorSubcoreMesh(core_axis_name="c", subcore_axis_name="s")


@functools.partial(
    pl.kernel,
    out_type=jax.ShapeDtypeStruct((NW, NP), jnp.float32),
    mesh=_mesh,
    compiler_params=pltpu.CompilerParams(needs_layout_passes=False),
    scratch_types=[
        pltpu.VMEM((NCHUNK, C), jnp.int32),
        pltpu.VMEM((NP,), jnp.float32),
    ],
)
def _sc_deg(dst_hbm, zerosflat_hbm, out_hbm, dst_v, deg_v):
    c = lax.axis_index("c")
    s = lax.axis_index("s")
    wid = c * NS + s
    pltpu.sync_copy(dst_hbm.at[wid], dst_v)
    pltpu.sync_copy(zerosflat_hbm, deg_v)
    ones16 = jnp.ones((16,), jnp.float32)

    def body(i, _):
        for t in range(C // 16):
            ids = dst_v[i, pl.ds(t * 16, 16)]
            plsc.addupdate_scatter(deg_v, [ids], ones16)
        return 0

    lax.fori_loop(0, NCHUNK, body, 0)
    pltpu.sync_copy(deg_v, out_hbm.at[wid])


IB = 5
NBLK = NCHUNK // IB


@functools.partial(
    pl.kernel,
    out_type=jax.ShapeDtypeStruct((NC, NP, D), jnp.float32),
    mesh=_mesh,
    scratch_types=[
        pltpu.VMEM((2, IB, C), jnp.int32),
        pltpu.VMEM((2, IB, C), jnp.int32),
        pltpu.VMEM((3, C, D), jnp.float32),
        pltpu.SemaphoreType.DMA((2,)),
        pltpu.SemaphoreType.DMA((3,)),
        pltpu.SemaphoreType.DMA((3,)),
        pltpu.VMEM_SHARED((NP, D), jnp.float32),
    ],
)
def _sc_hop(rst_hbm, src_hbm, dst_hbm, zeros_hbm, out_hbm,
            sidx_v, didx_v, rows_v, isem, gsem, ssem, agg_sp):
    c = lax.axis_index("c")
    s = lax.axis_index("s")
    wid = c * NS + s
    pltpu.sync_copy(zeros_hbm, agg_sp.at[pl.ds(s * RPW, RPW)])
    pltpu.sync_copy(src_hbm.at[wid, 0], sidx_v.at[0])
    pltpu.sync_copy(dst_hbm.at[wid, 0], didx_v.at[0])
    pltpu.async_copy(src_hbm.at[wid, 1], sidx_v.at[1], isem.at[1])
    pltpu.async_copy(dst_hbm.at[wid, 1], didx_v.at[1], isem.at[1])
    plsc.subcore_barrier()

    pltpu.async_copy(rst_hbm.at[sidx_v.at[0, 0]], rows_v.at[0], gsem.at[0])
    pltpu.async_copy(rst_hbm.at[sidx_v.at[0, 1]], rows_v.at[1], gsem.at[1])

    def body(i, _):
        b = i // IB
        r = lax.rem(i, IB)
        jb = lax.rem(b, 2)
        j = lax.rem(i, 3)
        pltpu.make_async_copy(rst_hbm.at[sidx_v.at[jb, r]], rows_v.at[j],
                              gsem.at[j]).wait()
        pltpu.async_copy(rows_v.at[j], agg_sp.at[didx_v.at[jb, r]],
                         ssem.at[j], add=True)

        @pl.when(jnp.logical_and(r == IB - 2, b + 1 < NBLK))
        def _():
            nb = lax.rem(b + 1, 2)
            pltpu.make_async_copy(src_hbm.at[wid, b + 1], sidx_v.at[nb],
                                  isem.at[nb]).wait()
            pltpu.make_async_copy(dst_hbm.at[wid, b + 1], didx_v.at[nb],
                                  isem.at[nb]).wait()

        @pl.when(jnp.logical_and(i >= 1, i + 2 < NCHUNK))
        def _():
            ip = i - 1
            j2 = lax.rem(i + 2, 3)
            pltpu.make_async_copy(
                rows_v.at[j2],
                agg_sp.at[didx_v.at[lax.rem(ip // IB, 2), lax.rem(ip, IB)]],
                ssem.at[j2]).wait()
            i2 = i + 2
            pltpu.async_copy(rst_hbm.at[sidx_v.at[lax.rem(i2 // IB, 2),
                                                  lax.rem(i2, IB)]],
                             rows_v.at[j2], gsem.at[j2])

        @pl.when(i == 0)
        def _():
            pltpu.async_copy(rst_hbm.at[sidx_v.at[0, 2]], rows_v.at[2],
                             gsem.at[2])

        @pl.when(jnp.logical_and(r == 0,
                                 jnp.logical_and(b >= 1, b + 1 < NBLK)))
        def _():
            nb = lax.rem(b + 1, 2)
            pltpu.async_copy(src_hbm.at[wid, b + 1], sidx_v.at[nb],
                             isem.at[nb])
            pltpu.async_copy(dst_hbm.at[wid, b + 1], didx_v.at[nb],
                             isem.at[nb])

        return 0

    lax.fori_loop(0, NCHUNK, body, 0)
    for k in (NCHUNK - 3, NCHUNK - 2, NCHUNK - 1):
        pltpu.make_async_copy(
            rows_v.at[k % 3],
            agg_sp.at[didx_v.at[(k // IB) % 2, k % IB]],
            ssem.at[k % 3]).wait()
    plsc.subcore_barrier()
    pltpu.sync_copy(agg_sp.at[pl.ds(s * RPW, RPW)],
                    out_hbm.at[c].at[pl.ds(s * RPW, RPW)])


DR = NP // D


def _degsum_body(dp_ref, out_ref):
    out_ref[...] = jnp.sum(dp_ref[...], axis=0)


def _tc_degsum(dp):
    return pl.pallas_call(
        _degsum_body,
        out_shape=jax.ShapeDtypeStruct((DR, D), jnp.float32),
    )(dp.reshape(NW, DR, D))


_R = 1000


def _prep_body(deg_ref, feat_ref, norm_ref, rst_ref):
    nrm = lax.rsqrt(jnp.maximum(deg_ref[...], 1.0))
    norm_ref[...] = nrm
    rst_ref[...] = feat_ref[...] * nrm


def _tc_prep(deg, feat):
    return pl.pallas_call(
        _prep_body,
        grid=(N // _R,),
        in_specs=[
            pl.BlockSpec((_R, 1), lambda i: (i, 0)),
            pl.BlockSpec((_R, D), lambda i: (i, 0)),
        ],
        out_specs=[
            pl.BlockSpec((_R, 1), lambda i: (i, 0)),
            pl.BlockSpec((_R, D), lambda i: (i, 0)),
        ],
        out_shape=[
            jax.ShapeDtypeStruct((N, 1), jnp.float32),
            jax.ShapeDtypeStruct((N, D), jnp.float32),
        ],
    )(deg, feat)


def _mid_body(p0_ref, p1_ref, norm_ref, h_ref, rst_ref):
    nrm = norm_ref[...]
    h = (p0_ref[0] + p1_ref[0]) * nrm
    h_ref[...] = h
    rst_ref[...] = h * nrm


def _tc_mid(p, norm):
    return pl.pallas_call(
        _mid_body,
        grid=(N // _R,),
        in_specs=[
            pl.BlockSpec((1, _R, D), lambda i: (0, i, 0)),
            pl.BlockSpec((1, _R, D), lambda i: (1, i, 0)),
            pl.BlockSpec((_R, 1), lambda i: (i, 0)),
        ],
        out_specs=[
            pl.BlockSpec((_R, D), lambda i: (i, 0)),
            pl.BlockSpec((_R, D), lambda i: (i, 0)),
        ],
        out_shape=[
            jax.ShapeDtypeStruct((N, D), jnp.float32),
            jax.ShapeDtypeStruct((N, D), jnp.float32),
        ],
    )(p, p, norm)


def _final_body(p0_ref, p1_ref, norm_ref, feat_ref, h1_ref,
                w0_ref, w1_ref, w2_ref, b_ref, out_ref):
    h2 = (p0_ref[0] + p1_ref[0]) * norm_ref[...]
    acc = jnp.dot(feat_ref[...], w0_ref[...],
                  preferred_element_type=jnp.float32)
    acc += jnp.dot(h1_ref[...], w1_ref[...],
                   preferred_element_type=jnp.float32)
    acc += jnp.dot(h2, w2_ref[...], preferred_element_type=jnp.float32)
    out_ref[...] = acc + b_ref[...]


def _tc_final(p, norm, feat, h1, w0, w1, w2, b2d):
    OUT = w0.shape[1]
    return pl.pallas_call(
        _final_body,
        grid=(N // _R,),
        in_specs=[
            pl.BlockSpec((1, _R, D), lambda i: (0, i, 0)),
            pl.BlockSpec((1, _R, D), lambda i: (1, i, 0)),
            pl.BlockSpec((_R, 1), lambda i: (i, 0)),
            pl.BlockSpec((_R, D), lambda i: (i, 0)),
            pl.BlockSpec((_R, D), lambda i: (i, 0)),
            pl.BlockSpec((D, OUT), lambda i: (0, 0)),
            pl.BlockSpec((D, OUT), lambda i: (0, 0)),
            pl.BlockSpec((D, OUT), lambda i: (0, 0)),
            pl.BlockSpec((1, OUT), lambda i: (0, 0)),
        ],
        out_specs=pl.BlockSpec((_R, OUT), lambda i: (i, 0)),
        out_shape=jax.ShapeDtypeStruct((N, OUT), jnp.float32),
    )(p, p, norm, feat, h1, w0, w1, w2, b2d)


def kernel(feat, edge_index, W, b):
    npad = EP - E
    pad_src = jnp.asarray(np.arange(npad, dtype=np.int32) % N)
    pad_dst = jnp.asarray(N + np.arange(npad, dtype=np.int32) % (NP - N))
    src = jnp.concatenate([edge_index[0], pad_src]).reshape(NW, NBLK, IB, C)
    dst = jnp.concatenate([edge_index[1], pad_dst]).reshape(NW, NBLK, IB, C)
    zerosflat = jnp.zeros((NP,), jnp.float32)
    zeros = jnp.zeros((RPW, D), jnp.float32)

    dp = _sc_deg(dst.reshape(NW, NCHUNK, C), zerosflat)
    deg = _tc_degsum(dp).reshape(NP, 1)[:N]
    norm, rst = _tc_prep(deg, feat)

    p = _sc_hop(rst, src, dst, zeros)
    h1, rst1 = _tc_mid(p, norm)

    p2 = _sc_hop(rst1, src, dst, zeros)

    w0, w1, w2 = W[:D], W[D:2 * D], W[2 * D:]
    return _tc_final(p2, norm, feat, h1, w0, w1, w2,
                     b.reshape(1, -1))


# speedup vs baseline: 1.0503x; 1.0015x over previous
import functools

import jax
import numpy as np
import jax.numpy as jnp
from jax import lax
from jax.experimental import pallas as pl
from jax.experimental.pallas import tpu as pltpu
from jax.experimental.pallas import tpu_sc as plsc

N = 10000
E = 320000
D = 128
K = 2

NC = 2
NS = 16
NW = NC * NS
EW = E // NW
C = 112
NCHUNK = 90
EWP = NCHUNK * C
EP = NW * EWP
NP = 10240
RPW = NP // NS
DEGW = 16

_mesh = plsc.VectorSubcoreMesh(core_axis_name="c", subcore_axis_name="s")


@functools.partial(
    pl.kernel,
    out_type=jax.ShapeDtypeStruct((NW, NP), jnp.float32),
    mesh=_mesh,
    compiler_params=pltpu.CompilerParams(needs_layout_passes=False),
    scratch_types=[
        pltpu.VMEM((EW,), jnp.int32),
        pltpu.VMEM((NP,), jnp.float32),
    ],
)
def _sc_deg(dst_hbm, zerosflat_hbm, out_hbm, dst_v, deg_v):
    c = lax.axis_index("c")
    s = lax.axis_index("s")
    wid = c * NS + s
    pltpu.sync_copy(dst_hbm.at[pl.ds(wid * EW, EW)], dst_v)
    pltpu.sync_copy(zerosflat_hbm, deg_v)
    ones16 = jnp.ones((16,), jnp.float32)

    def body(i, _):
        ids = dst_v[pl.ds(i * 16, 16)]
        plsc.addupdate_scatter(deg_v, [ids], ones16)
        return 0

    lax.fori_loop(0, EW // 16, body, 0)
    pltpu.sync_copy(deg_v, out_hbm.at[wid])


IB = 5
NBLK = NCHUNK // IB


@functools.partial(
    pl.kernel,
    out_type=jax.ShapeDtypeStruct((NC, NP, D), jnp.float32),
    mesh=_mesh,
    scratch_types=[
        pltpu.VMEM((2, IB, C), jnp.int32),
        pltpu.VMEM((2, IB, C), jnp.int32),
        pltpu.VMEM((3, C, D), jnp.float32),
        pltpu.SemaphoreType.DMA((2,)),
        pltpu.SemaphoreType.DMA((3,)),
        pltpu.SemaphoreType.DMA((3,)),
        pltpu.VMEM_SHARED((NP, D), jnp.float32),
    ],
)
def _sc_hop(rst_hbm, src_hbm, dst_hbm, zeros_hbm, out_hbm,
            sidx_v, didx_v, rows_v, isem, gsem, ssem, agg_sp):
    c = lax.axis_index("c")
    s = lax.axis_index("s")
    wid = c * NS + s
    pltpu.sync_copy(zeros_hbm, agg_sp.at[pl.ds(s * RPW, RPW)])
    pltpu.sync_copy(src_hbm.at[wid, 0], sidx_v.at[0])
    pltpu.sync_copy(dst_hbm.at[wid, 0], didx_v.at[0])
    pltpu.async_copy(src_hbm.at[wid, 1], sidx_v.at[1], isem.at[1])
    pltpu.async_copy(dst_hbm.at[wid, 1], didx_v.at[1], isem.at[1])
    plsc.subcore_barrier()

    pltpu.async_copy(rst_hbm.at[sidx_v.at[0, 0]], rows_v.at[0], gsem.at[0])
    pltpu.async_copy(rst_hbm.at[sidx_v.at[0, 1]], rows_v.at[1], gsem.at[1])

    def body(i, _):
        b = i // IB
        r = lax.rem(i, IB)
        jb = lax.rem(b, 2)
        j = lax.rem(i, 3)
        pltpu.make_async_copy(rst_hbm.at[sidx_v.at[jb, r]], rows_v.at[j],
                              gsem.at[j]).wait()
        pltpu.async_copy(rows_v.at[j], agg_sp.at[didx_v.at[jb, r]],
                         ssem.at[j], add=True)

        @pl.when(jnp.logical_and(r == IB - 2, b + 1 < NBLK))
        def _():
            nb = lax.rem(b + 1, 2)
            pltpu.make_async_copy(src_hbm.at[wid, b + 1], sidx_v.at[nb],
                                  isem.at[nb]).wait()
            pltpu.make_async_copy(dst_hbm.at[wid, b + 1], didx_v.at[nb],
                                  isem.at[nb]).wait()

        @pl.when(jnp.logical_and(i >= 1, i + 2 < NCHUNK))
        def _():
            ip = i - 1
            j2 = lax.rem(i + 2, 3)
            pltpu.make_async_copy(
                rows_v.at[j2],
                agg_sp.at[didx_v.at[lax.rem(ip // IB, 2), lax.rem(ip, IB)]],
                ssem.at[j2]).wait()
            i2 = i + 2
            pltpu.async_copy(rst_hbm.at[sidx_v.at[lax.rem(i2 // IB, 2),
                                                  lax.rem(i2, IB)]],
                             rows_v.at[j2], gsem.at[j2])

        @pl.when(i == 0)
        def _():
            pltpu.async_copy(rst_hbm.at[sidx_v.at[0, 2]], rows_v.at[2],
                             gsem.at[2])

        @pl.when(jnp.logical_and(r == 0,
                                 jnp.logical_and(b >= 1, b + 1 < NBLK)))
        def _():
            nb = lax.rem(b + 1, 2)
            pltpu.async_copy(src_hbm.at[wid, b + 1], sidx_v.at[nb],
                             isem.at[nb])
            pltpu.async_copy(dst_hbm.at[wid, b + 1], didx_v.at[nb],
                             isem.at[nb])

        return 0

    lax.fori_loop(0, NCHUNK, body, 0)
    for k in (NCHUNK - 3, NCHUNK - 2, NCHUNK - 1):
        pltpu.make_async_copy(
            rows_v.at[k % 3],
            agg_sp.at[didx_v.at[(k // IB) % 2, k % IB]],
            ssem.at[k % 3]).wait()
    plsc.subcore_barrier()
    pltpu.sync_copy(agg_sp.at[pl.ds(s * RPW, RPW)],
                    out_hbm.at[c].at[pl.ds(s * RPW, RPW)])


DR = NP // D


def _degsum_body(dp_ref, out_ref):
    out_ref[...] = jnp.sum(dp_ref[...], axis=0)


def _tc_degsum(dp):
    return pl.pallas_call(
        _degsum_body,
        out_shape=jax.ShapeDtypeStruct((DR, D), jnp.float32),
    )(dp.reshape(NW, DR, D))


_R = 1000


def _prep_body(deg_ref, feat_ref, norm_ref, rst_ref):
    nrm = lax.rsqrt(jnp.maximum(deg_ref[...], 1.0))
    norm_ref[...] = nrm
    rst_ref[...] = feat_ref[...] * nrm


def _tc_prep(deg, feat):
    return pl.pallas_call(
        _prep_body,
        grid=(N // _R,),
        in_specs=[
            pl.BlockSpec((_R, 1), lambda i: (i, 0)),
            pl.BlockSpec((_R, D), lambda i: (i, 0)),
        ],
        out_specs=[
            pl.BlockSpec((_R, 1), lambda i: (i, 0)),
            pl.BlockSpec((_R, D), lambda i: (i, 0)),
        ],
        out_shape=[
            jax.ShapeDtypeStruct((N, 1), jnp.float32),
            jax.ShapeDtypeStruct((N, D), jnp.float32),
        ],
    )(deg, feat)


def _mid_body(p0_ref, p1_ref, norm_ref, h_ref, rst_ref):
    nrm = norm_ref[...]
    h = (p0_ref[0] + p1_ref[0]) * nrm
    h_ref[...] = h
    rst_ref[...] = h * nrm


def _tc_mid(p, norm):
    return pl.pallas_call(
        _mid_body,
        grid=(N // _R,),
        in_specs=[
            pl.BlockSpec((1, _R, D), lambda i: (0, i, 0)),
            pl.BlockSpec((1, _R, D), lambda i: (1, i, 0)),
            pl.BlockSpec((_R, 1), lambda i: (i, 0)),
        ],
        out_specs=[
            pl.BlockSpec((_R, D), lambda i: (i, 0)),
            pl.BlockSpec((_R, D), lambda i: (i, 0)),
        ],
        out_shape=[
            jax.ShapeDtypeStruct((N, D), jnp.float32),
            jax.ShapeDtypeStruct((N, D), jnp.float32),
        ],
    )(p, p, norm)


def _final_body(p0_ref, p1_ref, norm_ref, feat_ref, h1_ref,
                w0_ref, w1_ref, w2_ref, b_ref, out_ref):
    h2 = (p0_ref[0] + p1_ref[0]) * norm_ref[...]
    acc = jnp.dot(feat_ref[...], w0_ref[...],
                  preferred_element_type=jnp.float32)
    acc += jnp.dot(h1_ref[...], w1_ref[...],
                   preferred_element_type=jnp.float32)
    acc += jnp.dot(h2, w2_ref[...], preferred_element_type=jnp.float32)
    out_ref[...] = acc + b_ref[...]


def _tc_final(p, norm, feat, h1, w0, w1, w2, b2d):
    OUT = w0.shape[1]
    return pl.pallas_call(
        _final_body,
        grid=(N // _R,),
        in_specs=[
            pl.BlockSpec((1, _R, D), lambda i: (0, i, 0)),
            pl.BlockSpec((1, _R, D), lambda i: (1, i, 0)),
            pl.BlockSpec((_R, 1), lambda i: (i, 0)),
            pl.BlockSpec((_R, D), lambda i: (i, 0)),
            pl.BlockSpec((_R, D), lambda i: (i, 0)),
            pl.BlockSpec((D, OUT), lambda i: (0, 0)),
            pl.BlockSpec((D, OUT), lambda i: (0, 0)),
            pl.BlockSpec((D, OUT), lambda i: (0, 0)),
            pl.BlockSpec((1, OUT), lambda i: (0, 0)),
        ],
        out_specs=pl.BlockSpec((_R, OUT), lambda i: (i, 0)),
        out_shape=jax.ShapeDtypeStruct((N, OUT), jnp.float32),
    )(p, p, norm, feat, h1, w0, w1, w2, b2d)


def kernel(feat, edge_index, W, b):
    npad = EP - E
    pad_src = jnp.asarray(np.arange(npad, dtype=np.int32) % N)
    pad_dst = jnp.asarray(N + np.arange(npad, dtype=np.int32) % (NP - N))
    src = jnp.concatenate([edge_index[0], pad_src]).reshape(NW, NBLK, IB, C)
    dst = jnp.concatenate([edge_index[1], pad_dst]).reshape(NW, NBLK, IB, C)
    zerosflat = jnp.zeros((NP,), jnp.float32)
    zeros = jnp.zeros((RPW, D), jnp.float32)

    dp = _sc_deg(edge_index[1], zerosflat)
    deg = _tc_degsum(dp).reshape(NP, 1)[:N]
    norm, rst = _tc_prep(deg, feat)

    p = _sc_hop(rst, src, dst, zeros)
    h1, rst1 = _tc_mid(p, norm)

    p2 = _sc_hop(rst1, src, dst, zeros)

    w0, w1, w2 = W[:D], W[D:2 * D], W[2 * D:]
    return _tc_final(p2, norm, feat, h1, w0, w1, w2,
                     b.reshape(1, -1))


# speedup vs baseline: 1.0760x; 1.0245x over previous
import functools

import jax
import numpy as np
import jax.numpy as jnp
from jax import lax
from jax.experimental import pallas as pl
from jax.experimental.pallas import tpu as pltpu
from jax.experimental.pallas import tpu_sc as plsc

N = 10000
E = 320000
D = 128
K = 2

NC = 2
NS = 16
NW = NC * NS
EW = E // NW
C = 112
NCHUNK = 90
EWP = NCHUNK * C
EP = NW * EWP
NP = 10240
RPW = NP // NS
DEGW = 16

_mesh = plsc.VectorSubcoreMesh(core_axis_name="c", subcore_axis_name="s")


@functools.partial(
    pl.kernel,
    out_type=jax.ShapeDtypeStruct((NW, NP), jnp.float32),
    mesh=_mesh,
    compiler_params=pltpu.CompilerParams(needs_layout_passes=False),
    scratch_types=[
        pltpu.VMEM((EW,), jnp.int32),
        pltpu.VMEM((NP,), jnp.float32),
    ],
)
def _sc_deg(dst_hbm, zerosflat_hbm, out_hbm, dst_v, deg_v):
    c = lax.axis_index("c")
    s = lax.axis_index("s")
    wid = c * NS + s
    pltpu.sync_copy(dst_hbm.at[pl.ds(wid * EW, EW)], dst_v)
    pltpu.sync_copy(zerosflat_hbm, deg_v)
    ones16 = jnp.ones((16,), jnp.float32)

    def body(i, _):
        ids = dst_v[pl.ds(i * 16, 16)]
        plsc.addupdate_scatter(deg_v, [ids], ones16)
        return 0

    lax.fori_loop(0, EW // 16, body, 0)
    pltpu.sync_copy(deg_v, out_hbm.at[wid])


IB = 5
NBLK = NCHUNK // IB


@functools.partial(
    pl.kernel,
    out_type=jax.ShapeDtypeStruct((NC, NP, D), jnp.float32),
    mesh=_mesh,
    scratch_types=[
        pltpu.VMEM((2, IB, C), jnp.int32),
        pltpu.VMEM((2, IB, C), jnp.int32),
        pltpu.VMEM((3, C, D), jnp.float32),
        pltpu.SemaphoreType.DMA((2,)),
        pltpu.SemaphoreType.DMA((3,)),
        pltpu.SemaphoreType.DMA((3,)),
        pltpu.VMEM_SHARED((NP, D), jnp.float32),
    ],
)
def _sc_hop(rst_hbm, src_hbm, dst_hbm, zeros_hbm, out_hbm,
            sidx_v, didx_v, rows_v, isem, gsem, ssem, agg_sp):
    c = lax.axis_index("c")
    s = lax.axis_index("s")
    wid = c * NS + s
    pltpu.sync_copy(zeros_hbm, agg_sp.at[pl.ds(s * RPW, RPW)])
    pltpu.sync_copy(src_hbm.at[wid, 0], sidx_v.at[0])
    pltpu.sync_copy(dst_hbm.at[wid, 0], didx_v.at[0])
    pltpu.async_copy(src_hbm.at[wid, 1], sidx_v.at[1], isem.at[1])
    pltpu.async_copy(dst_hbm.at[wid, 1], didx_v.at[1], isem.at[1])
    plsc.subcore_barrier()

    pltpu.async_copy(rst_hbm.at[sidx_v.at[0, 0]], rows_v.at[0], gsem.at[0])
    pltpu.async_copy(rst_hbm.at[sidx_v.at[0, 1]], rows_v.at[1], gsem.at[1])

    def body(i, _):
        b = i // IB
        r = lax.rem(i, IB)
        jb = lax.rem(b, 2)
        j = lax.rem(i, 3)
        pltpu.make_async_copy(rst_hbm.at[sidx_v.at[jb, r]], rows_v.at[j],
                              gsem.at[j]).wait()
        pltpu.async_copy(rows_v.at[j], agg_sp.at[didx_v.at[jb, r]],
                         ssem.at[j], add=True)

        @pl.when(jnp.logical_and(r == IB - 2, b + 1 < NBLK))
        def _():
            nb = lax.rem(b + 1, 2)
            pltpu.make_async_copy(src_hbm.at[wid, b + 1], sidx_v.at[nb],
                                  isem.at[nb]).wait()
            pltpu.make_async_copy(dst_hbm.at[wid, b + 1], didx_v.at[nb],
                                  isem.at[nb]).wait()

        @pl.when(jnp.logical_and(i >= 1, i + 2 < NCHUNK))
        def _():
            ip = i - 1
            j2 = lax.rem(i + 2, 3)
            pltpu.make_async_copy(
                rows_v.at[j2],
                agg_sp.at[didx_v.at[lax.rem(ip // IB, 2), lax.rem(ip, IB)]],
                ssem.at[j2]).wait()
            i2 = i + 2
            pltpu.async_copy(rst_hbm.at[sidx_v.at[lax.rem(i2 // IB, 2),
                                                  lax.rem(i2, IB)]],
                             rows_v.at[j2], gsem.at[j2])

        @pl.when(i == 0)
        def _():
            pltpu.async_copy(rst_hbm.at[sidx_v.at[0, 2]], rows_v.at[2],
                             gsem.at[2])

        @pl.when(jnp.logical_and(r == 0,
                                 jnp.logical_and(b >= 1, b + 1 < NBLK)))
        def _():
            nb = lax.rem(b + 1, 2)
            pltpu.async_copy(src_hbm.at[wid, b + 1], sidx_v.at[nb],
                             isem.at[nb])
            pltpu.async_copy(dst_hbm.at[wid, b + 1], didx_v.at[nb],
                             isem.at[nb])

        return 0

    lax.fori_loop(0, NCHUNK, body, 0)
    for k in (NCHUNK - 3, NCHUNK - 2, NCHUNK - 1):
        pltpu.make_async_copy(
            rows_v.at[k % 3],
            agg_sp.at[didx_v.at[(k // IB) % 2, k % IB]],
            ssem.at[k % 3]).wait()
    plsc.subcore_barrier()
    pltpu.sync_copy(agg_sp.at[pl.ds(s * RPW, RPW)],
                    out_hbm.at[c].at[pl.ds(s * RPW, RPW)])


DR = NP // D


def _degsum_body(dp_ref, out_ref):
    out_ref[...] = jnp.sum(dp_ref[...], axis=0)


def _tc_degsum(dp):
    return pl.pallas_call(
        _degsum_body,
        out_shape=jax.ShapeDtypeStruct((DR, D), jnp.float32),
    )(dp.reshape(NW, DR, D))


_R = 2000


def _prep_body(deg_ref, feat_ref, norm_ref, rst_ref):
    nrm = lax.rsqrt(jnp.maximum(deg_ref[...], 1.0))
    norm_ref[...] = nrm
    rst_ref[...] = feat_ref[...] * nrm


def _tc_prep(deg, feat):
    return pl.pallas_call(
        _prep_body,
        grid=(N // _R,),
        in_specs=[
            pl.BlockSpec((_R, 1), lambda i: (i, 0)),
            pl.BlockSpec((_R, D), lambda i: (i, 0)),
        ],
        out_specs=[
            pl.BlockSpec((_R, 1), lambda i: (i, 0)),
            pl.BlockSpec((_R, D), lambda i: (i, 0)),
        ],
        out_shape=[
            jax.ShapeDtypeStruct((N, 1), jnp.float32),
            jax.ShapeDtypeStruct((N, D), jnp.float32),
        ],
    )(deg, feat)


def _mid_body(p0_ref, p1_ref, norm_ref, h_ref, rst_ref):
    nrm = norm_ref[...]
    h = (p0_ref[0] + p1_ref[0]) * nrm
    h_ref[...] = h
    rst_ref[...] = h * nrm


def _tc_mid(p, norm):
    return pl.pallas_call(
        _mid_body,
        grid=(N // _R,),
        in_specs=[
            pl.BlockSpec((1, _R, D), lambda i: (0, i, 0)),
            pl.BlockSpec((1, _R, D), lambda i: (1, i, 0)),
            pl.BlockSpec((_R, 1), lambda i: (i, 0)),
        ],
        out_specs=[
            pl.BlockSpec((_R, D), lambda i: (i, 0)),
            pl.BlockSpec((_R, D), lambda i: (i, 0)),
        ],
        out_shape=[
            jax.ShapeDtypeStruct((N, D), jnp.float32),
            jax.ShapeDtypeStruct((N, D), jnp.float32),
        ],
    )(p, p, norm)


def _final_body(p0_ref, p1_ref, norm_ref, feat_ref, h1_ref,
                w0_ref, w1_ref, w2_ref, b_ref, out_ref):
    h2 = (p0_ref[0] + p1_ref[0]) * norm_ref[...]
    acc = jnp.dot(feat_ref[...], w0_ref[...],
                  preferred_element_type=jnp.float32)
    acc += jnp.dot(h1_ref[...], w1_ref[...],
                   preferred_element_type=jnp.float32)
    acc += jnp.dot(h2, w2_ref[...], preferred_element_type=jnp.float32)
    out_ref[...] = acc + b_ref[...]


def _tc_final(p, norm, feat, h1, w0, w1, w2, b2d):
    OUT = w0.shape[1]
    return pl.pallas_call(
        _final_body,
        grid=(N // _R,),
        in_specs=[
            pl.BlockSpec((1, _R, D), lambda i: (0, i, 0)),
            pl.BlockSpec((1, _R, D), lambda i: (1, i, 0)),
            pl.BlockSpec((_R, 1), lambda i: (i, 0)),
            pl.BlockSpec((_R, D), lambda i: (i, 0)),
            pl.BlockSpec((_R, D), lambda i: (i, 0)),
            pl.BlockSpec((D, OUT), lambda i: (0, 0)),
            pl.BlockSpec((D, OUT), lambda i: (0, 0)),
            pl.BlockSpec((D, OUT), lambda i: (0, 0)),
            pl.BlockSpec((1, OUT), lambda i: (0, 0)),
        ],
        out_specs=pl.BlockSpec((_R, OUT), lambda i: (i, 0)),
        out_shape=jax.ShapeDtypeStruct((N, OUT), jnp.float32),
    )(p, p, norm, feat, h1, w0, w1, w2, b2d)


def kernel(feat, edge_index, W, b):
    npad = EP - E
    pad_src = jnp.asarray(np.arange(npad, dtype=np.int32) % N)
    pad_dst = jnp.asarray(N + np.arange(npad, dtype=np.int32) % (NP - N))
    src = jnp.concatenate([edge_index[0], pad_src]).reshape(NW, NBLK, IB, C)
    dst = jnp.concatenate([edge_index[1], pad_dst]).reshape(NW, NBLK, IB, C)
    zerosflat = jnp.zeros((NP,), jnp.float32)
    zeros = jnp.zeros((RPW, D), jnp.float32)

    dp = _sc_deg(edge_index[1], zerosflat)
    deg = _tc_degsum(dp).reshape(NP, 1)[:N]
    norm, rst = _tc_prep(deg, feat)

    p = _sc_hop(rst, src, dst, zeros)
    h1, rst1 = _tc_mid(p, norm)

    p2 = _sc_hop(rst1, src, dst, zeros)

    w0, w1, w2 = W[:D], W[D:2 * D], W[2 * D:]
    return _tc_final(p2, norm, feat, h1, w0, w1, w2,
                     b.reshape(1, -1))
